# baseline (speedup 1.0000x reference)
import functools

import jax
import jax.numpy as jnp
from jax import lax
from jax.experimental import pallas as pl
from jax.experimental.pallas import tpu as pltpu
from jax.experimental.pallas import tpu_sc as plsc

VOCAB_N = 100000
EMB_N = 300
HID_N = 64
NCLS_N = 10
B_N = 1024
S_N = 200
W_N = 128


_PROJ_ROWS = 4000


def _proj_body(emb_ref, wx_ref, b_ref, out_ref):
    out_ref[...] = (
        jnp.dot(emb_ref[...], wx_ref[...], preferred_element_type=jnp.float32)
        + b_ref[...]
    )


def _project_table(emb_table, Wx_pad, b_pad):
    v = emb_table.shape[0]
    grid = v // _PROJ_ROWS
    return pl.pallas_call(
        _proj_body,
        grid=(grid,),
        in_specs=[
            pl.BlockSpec((_PROJ_ROWS, EMB_N), lambda i: (i, 0)),
            pl.BlockSpec((EMB_N, W_N), lambda i: (0, 0)),
            pl.BlockSpec((1, W_N), lambda i: (0, 0)),
        ],
        out_specs=pl.BlockSpec((_PROJ_ROWS, W_N), lambda i: (i, 0)),
        out_shape=jax.ShapeDtypeStruct((v, W_N), jnp.float32),
    )(emb_table, Wx_pad, b_pad)


_GATHER_CHUNK = 640


def _make_gather(n_rows):
    info = plsc.get_sparse_core_info()
    nc, ns = info.num_cores, info.num_subcores
    nw = nc * ns
    per_w = n_rows // nw
    n_chunks = per_w // _GATHER_CHUNK
    assert per_w % _GATHER_CHUNK == 0

    mesh = plsc.VectorSubcoreMesh(core_axis_name="c", subcore_axis_name="s")

    @functools.partial(
        pl.kernel,
        mesh=mesh,
        out_type=jax.ShapeDtypeStruct((n_rows, W_N), jnp.float32),
        scratch_types=[
            pltpu.VMEM((_GATHER_CHUNK,), jnp.int32),
            pltpu.VMEM((_GATHER_CHUNK, W_N), jnp.float32),
            pltpu.SemaphoreType.DMA,
        ],
    )
    def gather_kernel(idx_hbm, table_hbm, out_hbm, idx_v, rows_v, sem):
        wid = lax.axis_index("s") * nc + lax.axis_index("c")
        base = wid * per_w
        for j in range(n_chunks):
            off = base + j * _GATHER_CHUNK
            pltpu.sync_copy(idx_hbm.at[pl.ds(off, _GATHER_CHUNK)], idx_v)
            pltpu.async_copy(table_hbm.at[idx_v], rows_v, sem).wait()
            pltpu.sync_copy(rows_v, out_hbm.at[pl.ds(off, _GATHER_CHUNK)])

    return gather_kernel


_B_BLK = 128
_T_BLK = S_N


def _rnn_body(x_ref, wh_ref, wdt_ref, bdt_ref, out_ref, hs_ref):
    wh = wh_ref[...]

    def step(i, h):
        u = x_ref[i]
        h = jnp.tanh(u + jnp.dot(h, wh, preferred_element_type=jnp.float32))
        hs_ref[pl.ds(i, 1)] = h[None]
        return h

    lax.fori_loop(0, _T_BLK, step, jnp.zeros((_B_BLK, W_N), jnp.float32))

    hs = hs_ref[...].reshape(_T_BLK * _B_BLK, W_N)
    logits_t = lax.dot_general(
        wdt_ref[...], hs, (((1,), (1,)), ((), ())),
        preferred_element_type=jnp.float32,
    )
    logits_t = logits_t + bdt_ref[...]
    row = lax.broadcasted_iota(jnp.int32, (16, _T_BLK * _B_BLK), 0)
    e = jnp.where(row < NCLS_N, jnp.exp(logits_t), 0.0)
    p = e * (1.0 / jnp.sum(e, axis=0, keepdims=True))
    out_ref[...] = p.reshape(16, _T_BLK, _B_BLK)


def _run_rnn(x_t, Wh_pad, WdT, bdT):
    nb = B_N // _B_BLK
    return pl.pallas_call(
        _rnn_body,
        grid=(nb,),
        in_specs=[
            pl.BlockSpec((_T_BLK, _B_BLK, W_N), lambda b: (0, b, 0)),
            pl.BlockSpec((W_N, W_N), lambda b: (0, 0)),
            pl.BlockSpec((16, W_N), lambda b: (0, 0)),
            pl.BlockSpec((16, 1), lambda b: (0, 0)),
        ],
        out_specs=pl.BlockSpec((16, _T_BLK, _B_BLK), lambda b: (0, 0, b)),
        out_shape=jax.ShapeDtypeStruct((16, S_N, B_N), jnp.float32),
        scratch_shapes=[
            pltpu.VMEM((_T_BLK, _B_BLK, W_N), jnp.float32),
        ],
    )(x_t, Wh_pad, WdT, bdT)


def kernel(input_tensor, emb_table, Wx, Wh, b_rnn, Wd, bd):
    tokens = input_tensor.astype(jnp.int32)
    b, s = tokens.shape

    Wx_pad = jnp.zeros((EMB_N, W_N), jnp.float32).at[:, :HID_N].set(Wx)
    b_pad = jnp.zeros((1, W_N), jnp.float32).at[:, :HID_N].set(b_rnn)
    Wh_pad = jnp.zeros((W_N, W_N), jnp.float32).at[:HID_N, :HID_N].set(Wh)
    WdT = jnp.zeros((16, W_N), jnp.float32).at[:NCLS_N, :HID_N].set(Wd.T)
    bdT = jnp.zeros((16, 1), jnp.float32).at[:NCLS_N, 0].set(bd)

    P = _project_table(emb_table, Wx_pad, b_pad)

    idx = jnp.swapaxes(tokens, 0, 1).reshape(-1)
    X = _make_gather(b * s)(idx, P)
    x_t = X.reshape(s, b, W_N)

    out_t = _run_rnn(x_t, Wh_pad, WdT, bdT)
    return out_t[:NCLS_N].transpose(2, 1, 0)


# speedup vs baseline: 1.2165x; 1.2165x over previous
import functools

import jax
import jax.numpy as jnp
from jax import lax
from jax.experimental import pallas as pl
from jax.experimental.pallas import tpu as pltpu
from jax.experimental.pallas import tpu_sc as plsc

VOCAB_N = 100000
EMB_N = 300
HID_N = 64
NCLS_N = 10
B_N = 1024
S_N = 200
W_N = 128


_PROJ_ROWS = 4000


def _proj_body(emb_ref, wx_ref, b_ref, out_ref):
    out_ref[...] = (
        jnp.dot(emb_ref[...], wx_ref[...], preferred_element_type=jnp.float32)
        + b_ref[...]
    )


def _project_table(emb_table, Wx_pad, b_pad):
    v = emb_table.shape[0]
    grid = v // _PROJ_ROWS
    return pl.pallas_call(
        _proj_body,
        grid=(grid,),
        in_specs=[
            pl.BlockSpec((_PROJ_ROWS, EMB_N), lambda i: (i, 0)),
            pl.BlockSpec((EMB_N, W_N), lambda i: (0, 0)),
            pl.BlockSpec((1, W_N), lambda i: (0, 0)),
        ],
        out_specs=pl.BlockSpec((_PROJ_ROWS, W_N), lambda i: (i, 0)),
        out_shape=jax.ShapeDtypeStruct((v, W_N), jnp.float32),
    )(emb_table, Wx_pad, b_pad)


_GATHER_CHUNK = 640


def _make_gather(n_rows):
    info = plsc.get_sparse_core_info()
    nc, ns = info.num_cores, info.num_subcores
    nw = nc * ns
    per_w = n_rows // nw
    n_chunks = per_w // _GATHER_CHUNK
    assert per_w % _GATHER_CHUNK == 0

    mesh = plsc.VectorSubcoreMesh(core_axis_name="c", subcore_axis_name="s")

    @functools.partial(
        pl.kernel,
        mesh=mesh,
        out_type=jax.ShapeDtypeStruct((n_rows, W_N), jnp.float32),
        scratch_types=[
            pltpu.VMEM((_GATHER_CHUNK,), jnp.int32),
            pltpu.VMEM((_GATHER_CHUNK, W_N), jnp.float32),
            pltpu.SemaphoreType.DMA,
        ],
    )
    def gather_kernel(idx_hbm, table_hbm, out_hbm, idx_v, rows_v, sem):
        wid = lax.axis_index("s") * nc + lax.axis_index("c")
        base = wid * per_w
        for j in range(n_chunks):
            off = base + j * _GATHER_CHUNK
            pltpu.sync_copy(idx_hbm.at[pl.ds(off, _GATHER_CHUNK)], idx_v)
            pltpu.async_copy(table_hbm.at[idx_v], rows_v, sem).wait()
            pltpu.sync_copy(rows_v, out_hbm.at[pl.ds(off, _GATHER_CHUNK)])

    return gather_kernel


_B_BLK = 256
_T_BLK = 40


def _rnn_body(x_ref, wh_ref, wdt_ref, bdt_ref, out_ref, h_ref, hs_ref):
    t_blk = pl.program_id(1)

    @pl.when(t_blk == 0)
    def _init():
        h_ref[...] = jnp.zeros_like(h_ref)

    wh = wh_ref[...]

    def step(i, h):
        u = x_ref[i]
        h = jnp.tanh(u + jnp.dot(h, wh, preferred_element_type=jnp.float32))
        hs_ref[pl.ds(i, 1)] = h[None]
        return h

    h_ref[...] = lax.fori_loop(0, _T_BLK, step, h_ref[...])

    hs = hs_ref[...].reshape(_T_BLK * _B_BLK, W_N)
    logits_t = lax.dot_general(
        wdt_ref[...], hs, (((1,), (1,)), ((), ())),
        preferred_element_type=jnp.float32,
    )
    logits_t = logits_t + bdt_ref[...]
    row = lax.broadcasted_iota(jnp.int32, (16, _T_BLK * _B_BLK), 0)
    e = jnp.where(row < NCLS_N, jnp.exp(logits_t), 0.0)
    p = e * (1.0 / jnp.sum(e, axis=0, keepdims=True))
    out_ref[:, pl.ds(t_blk * _T_BLK, _T_BLK), :] = p.reshape(16, _T_BLK, _B_BLK)


def _run_rnn(x_t, Wh_pad, WdT, bdT):
    nb = B_N // _B_BLK
    nt = S_N // _T_BLK
    return pl.pallas_call(
        _rnn_body,
        grid=(nb, nt),
        in_specs=[
            pl.BlockSpec((_T_BLK, _B_BLK, W_N), lambda b, t: (t, b, 0)),
            pl.BlockSpec((W_N, W_N), lambda b, t: (0, 0)),
            pl.BlockSpec((16, W_N), lambda b, t: (0, 0)),
            pl.BlockSpec((16, 1), lambda b, t: (0, 0)),
        ],
        out_specs=pl.BlockSpec((16, S_N, _B_BLK), lambda b, t: (0, 0, b)),
        out_shape=jax.ShapeDtypeStruct((16, S_N, B_N), jnp.float32),
        scratch_shapes=[
            pltpu.VMEM((_B_BLK, W_N), jnp.float32),
            pltpu.VMEM((_T_BLK, _B_BLK, W_N), jnp.float32),
        ],
    )(x_t, Wh_pad, WdT, bdT)


def kernel(input_tensor, emb_table, Wx, Wh, b_rnn, Wd, bd):
    tokens = input_tensor.astype(jnp.int32)
    b, s = tokens.shape

    Wx_pad = jnp.zeros((EMB_N, W_N), jnp.float32).at[:, :HID_N].set(Wx)
    b_pad = jnp.zeros((1, W_N), jnp.float32).at[:, :HID_N].set(b_rnn)
    Wh_pad = jnp.zeros((W_N, W_N), jnp.float32).at[:HID_N, :HID_N].set(Wh)
    WdT = jnp.zeros((16, W_N), jnp.float32).at[:NCLS_N, :HID_N].set(Wd.T)
    bdT = jnp.zeros((16, 1), jnp.float32).at[:NCLS_N, 0].set(bd)

    P = _project_table(emb_table, Wx_pad, b_pad)

    idx = jnp.swapaxes(tokens, 0, 1).reshape(-1)
    X = _make_gather(b * s)(idx, P)
    x_t = X.reshape(s, b, W_N)

    out_t = _run_rnn(x_t, Wh_pad, WdT, bdT)
    return out_t[:NCLS_N].transpose(2, 1, 0)
